# baseline (speedup 1.0000x reference)
import jax
import jax.numpy as jnp
from jax.experimental import pallas as pl
from jax.experimental.pallas import tpu as pltpu

N_POINTS = 100000
HIDDEN = 128
N_EIGEN = 5
BLOCK = 4000


def _mlp_kernel(t_ref, w1_ref, b1_ref, w2_ref, b2_ref, w3_ref, b3_ref, o_ref):
    t = t_ref[...]
    h = jnp.maximum(t * w1_ref[...] + b1_ref[...], 0.0)
    h = jnp.dot(h, w2_ref[...], preferred_element_type=jnp.float32)
    h = jnp.maximum(h + b2_ref[...], 0.0)
    o = jnp.dot(h, w3_ref[...], preferred_element_type=jnp.float32)
    o_ref[...] = o + b3_ref[...]


def kernel(t_values, W1, b1, W2, b2, W3, b3):
    t2d = t_values.reshape(N_POINTS, 1)
    b1r = b1.reshape(1, HIDDEN)
    b2r = b2.reshape(1, HIDDEN)
    b3r = b3.reshape(1, N_EIGEN)
    grid = (N_POINTS // BLOCK,)
    rep = lambda shape: pl.BlockSpec(shape, lambda i: (0, 0))
    return pl.pallas_call(
        _mlp_kernel,
        grid=grid,
        in_specs=[
            pl.BlockSpec((BLOCK, 1), lambda i: (i, 0)),
            rep((1, HIDDEN)),
            rep((1, HIDDEN)),
            rep((HIDDEN, HIDDEN)),
            rep((1, HIDDEN)),
            rep((HIDDEN, N_EIGEN)),
            rep((1, N_EIGEN)),
        ],
        out_specs=pl.BlockSpec((BLOCK, N_EIGEN), lambda i: (i, 0)),
        out_shape=jax.ShapeDtypeStruct((N_POINTS, N_EIGEN), jnp.float32),
        compiler_params=pltpu.CompilerParams(
            dimension_semantics=("arbitrary",),
        ),
    )(t2d, W1, b1r, W2, b2r, W3, b3r)


# speedup vs baseline: 3.2270x; 3.2270x over previous
import jax
import jax.numpy as jnp
from jax.experimental import pallas as pl
from jax.experimental.pallas import tpu as pltpu

N_POINTS = 100000
HIDDEN = 128
N_EIGEN = 5
N_PAD = 102400
BLOCK = 4096


def _mlp_kernel(t_ref, w1_ref, b1_ref, w2_ref, b2_ref, w3_ref, b3_ref, o_ref):
    t = t_ref[...]
    h = jnp.maximum(w1_ref[...] * t + b1_ref[...], 0.0)
    h = jnp.dot(w2_ref[...], h, preferred_element_type=jnp.float32)
    h = jnp.maximum(h + b2_ref[...], 0.0)
    o = jnp.dot(w3_ref[...], h, preferred_element_type=jnp.float32)
    o_ref[...] = o + b3_ref[...]


def kernel(t_values, W1, b1, W2, b2, W3, b3):
    t_row = jnp.pad(t_values, (0, N_PAD - N_POINTS)).reshape(1, N_PAD)
    w1c = W1.reshape(HIDDEN, 1)
    b1c = b1.reshape(HIDDEN, 1)
    w2t = W2.T
    b2c = b2.reshape(HIDDEN, 1)
    w3t = W3.T
    b3c = b3.reshape(N_EIGEN, 1)
    grid = (N_PAD // BLOCK,)
    rep = lambda shape: pl.BlockSpec(shape, lambda i: (0, 0))
    out_t = pl.pallas_call(
        _mlp_kernel,
        grid=grid,
        in_specs=[
            pl.BlockSpec((1, BLOCK), lambda i: (0, i)),
            rep((HIDDEN, 1)),
            rep((HIDDEN, 1)),
            rep((HIDDEN, HIDDEN)),
            rep((HIDDEN, 1)),
            rep((N_EIGEN, HIDDEN)),
            rep((N_EIGEN, 1)),
        ],
        out_specs=pl.BlockSpec((N_EIGEN, BLOCK), lambda i: (0, i)),
        out_shape=jax.ShapeDtypeStruct((N_EIGEN, N_PAD), jnp.float32),
        compiler_params=pltpu.CompilerParams(
            dimension_semantics=("arbitrary",),
        ),
    )(t_row, w1c, b1c, w2t, b2c, w3t, b3c)
    return out_t[:, :N_POINTS].T
